# baseline (speedup 1.0000x reference)
import functools

import jax
import jax.numpy as jnp
from jax import lax
from jax.experimental import pallas as pl
from jax.experimental.pallas import tpu as pltpu
from jax.experimental.pallas import tpu_sc as plsc

_NC = 2
_NS = 16
_LANES = 16


@functools.partial(jax.jit, static_argnums=(3, 4, 5))
def _segment_partials(zp, src, dst, NPAD, E, W):
    NW = _NC * _NS
    EW = E // NW
    NWIN = EW // W
    assert EW * NW == E and NWIN * W == EW and W % 8 == 0
    SL = NPAD // _NS
    assert SL * _NS == NPAD and SL % 128 == 0

    assert W % (2 * _LANES) == 0 and NWIN % 2 == 0

    mesh = plsc.VectorSubcoreMesh(core_axis_name="c", subcore_axis_name="s")

    @functools.partial(
        pl.kernel,
        out_type=jax.ShapeDtypeStruct((_NC, NPAD), jnp.float32),
        mesh=mesh,
        compiler_params=pltpu.CompilerParams(
            use_tc_tiling_on_sc=False, needs_layout_passes=False),
        scratch_types=[
            pltpu.VMEM_SHARED((NPAD,), jnp.float32),
            pltpu.VMEM((NPAD,), jnp.float32),
            [pltpu.VMEM((W,), jnp.int32)] * 2,
            [pltpu.VMEM((W,), jnp.int32)] * 2,
            [pltpu.VMEM((W,), jnp.float32)] * 2,
            [pltpu.SemaphoreType.DMA] * 2,
        ],
    )
    def seg(z_hbm, src_hbm, dst_hbm, out_hbm,
            acc_sh, z_t, dst_v, src_v, val_v, sems):
        cid = lax.axis_index("c")
        sid = lax.axis_index("s")

        pltpu.sync_copy(z_hbm, z_t)

        def zbody(i, carry):
            val_v[0][pl.ds(i * _LANES, _LANES)] = jnp.zeros((_LANES,), jnp.float32)
            return carry

        lax.fori_loop(0, W // _LANES, zbody, 0)
        full, rem = divmod(SL, W)
        for q in range(full):
            pltpu.sync_copy(val_v[0], acc_sh.at[pl.ds(sid * SL + q * W, W)])
        if rem:
            pltpu.sync_copy(val_v[0].at[pl.ds(0, rem)],
                            acc_sh.at[pl.ds(sid * SL + full * W, rem)])

        plsc.subcore_barrier()

        ebase = (cid * _NS + sid) * EW

        def window(i, h, first):
            off = ebase + i * W
            if not first:
                pltpu.make_async_copy(
                    val_v[h], acc_sh.at[src_v[h]], sems[h]).wait()
            pltpu.sync_copy(dst_hbm.at[pl.ds(off, W)], dst_v[h])
            pltpu.sync_copy(src_hbm.at[pl.ds(off, W)], src_v[h])

            def gbody(k, carry):
                idx = dst_v[h][pl.ds(k * _LANES, _LANES)]
                val_v[h][pl.ds(k * _LANES, _LANES)] = plsc.load_gather(
                    z_t, [idx])
                return carry

            lax.fori_loop(0, W // _LANES, gbody, 0)
            pltpu.async_copy(val_v[h], acc_sh.at[src_v[h]], sems[h], add=True)

        window(0, 0, True)
        window(1, 1, True)

        def body(j, carry):
            window(2 * j, 0, False)
            window(2 * j + 1, 1, False)
            return carry

        lax.fori_loop(1, NWIN // 2, body, 0)

        for h in range(2):
            pltpu.make_async_copy(val_v[h], acc_sh.at[src_v[h]], sems[h]).wait()

        plsc.subcore_barrier()

        @pl.when(sid == 0)
        def _writeout():
            pltpu.sync_copy(acc_sh, out_hbm.at[cid])

    return seg(zp, src, dst)


def _combine_body(s_ref, z_ref, d_ref, p_ref, o_ref):
    alpha = s_ref[0]
    beta = s_ref[1]
    gamma = s_ref[2]
    bias = s_ref[3]
    logd = jnp.log(d_ref[...])
    az = p_ref[0] + p_ref[1]
    o_ref[...] = (alpha * jnp.exp(gamma * logd) * z_ref[...]
                  + beta * jnp.exp((gamma - 1.0) * logd) * az + bias)


def kernel(z, edge_index, D, params):
    N = z.shape[0]
    E = edge_index.shape[1]
    src = edge_index[0]
    dst = edge_index[1]

    NPAD = -(-N // (_NS * 128)) * (_NS * 128)
    zp = jnp.pad(z, (0, NPAD - N))
    dp = jnp.pad(D, (0, NPAD - N), constant_values=1.0)

    partial = _segment_partials(zp, src, dst, NPAD, E, 4000)

    alpha = jnp.exp(params[0])
    beta = -alpha * jnp.exp(params[1])
    gamma = jnp.exp(params[2])
    scal = jnp.stack([alpha, beta, gamma, params[3]])

    combine = pl.pallas_call(
        _combine_body,
        out_shape=jax.ShapeDtypeStruct((NPAD,), jnp.float32),
        in_specs=[
            pl.BlockSpec(memory_space=pltpu.SMEM),
            pl.BlockSpec(memory_space=pltpu.VMEM),
            pl.BlockSpec(memory_space=pltpu.VMEM),
            pl.BlockSpec(memory_space=pltpu.VMEM),
        ],
        out_specs=pl.BlockSpec(memory_space=pltpu.VMEM),
    )
    return combine(scal, zp, dp, partial)[:N]


# speedup vs baseline: 1.5953x; 1.5953x over previous
import functools

import jax
import jax.numpy as jnp
from jax import lax
from jax.experimental import pallas as pl
from jax.experimental.pallas import tpu as pltpu
from jax.experimental.pallas import tpu_sc as plsc

_NC = 2
_NS = 16
_LANES = 16


@functools.partial(jax.jit, static_argnums=(3, 4, 5))
def _segment_partials(z, src, dst, N, E, W):
    NW = _NC * _NS
    EW = E // NW
    NWIN = EW // W
    assert EW * NW == E and NWIN * W == EW
    assert W % (2 * _LANES) == 0 and NWIN % 2 == 0 and NWIN >= 4
    SL = -(-N // (_NS * 8)) * 8
    NPAD = SL * _NS

    mesh = plsc.VectorSubcoreMesh(core_axis_name="c", subcore_axis_name="s")

    @functools.partial(
        pl.kernel,
        out_type=jax.ShapeDtypeStruct((_NC, N), jnp.float32),
        mesh=mesh,
        compiler_params=pltpu.CompilerParams(
            use_tc_tiling_on_sc=False, needs_layout_passes=False),
        scratch_types=[
            pltpu.VMEM_SHARED((NPAD,), jnp.float32),
            pltpu.VMEM((N,), jnp.float32),
            [pltpu.VMEM((W,), jnp.int32)] * 2,
            [pltpu.VMEM((W,), jnp.int32)] * 2,
            [pltpu.VMEM((W,), jnp.float32)] * 2,
            [pltpu.SemaphoreType.DMA] * 2,
            [pltpu.SemaphoreType.DMA] * 2,
            pltpu.SemaphoreType.DMA,
        ],
    )
    def seg(z_hbm, src_hbm, dst_hbm, out_hbm,
            acc_sh, z_t, dst_v, src_v, val_v, dsem, csem, ssem):
        cid = lax.axis_index("c")
        sid = lax.axis_index("s")
        ebase = (cid * _NS + sid) * EW

        def zbody(i, carry):
            val_v[0][pl.ds(i * _LANES, _LANES)] = jnp.zeros((_LANES,), jnp.float32)
            return carry

        lax.fori_loop(0, W // _LANES, zbody, 0)
        full, rem = divmod(SL, W)
        for q in range(full):
            pltpu.sync_copy(val_v[0], acc_sh.at[pl.ds(sid * SL + q * W, W)])
        if rem:
            pltpu.sync_copy(val_v[0].at[pl.ds(0, rem)],
                            acc_sh.at[pl.ds(sid * SL + full * W, rem)])

        pltpu.sync_copy(z_hbm, z_t)

        plsc.subcore_barrier()

        def fetch_dst(i, s):
            pltpu.async_copy(dst_hbm.at[pl.ds(ebase + i * W, W)],
                             dst_v[s], dsem[s])

        def window(i, s, first):
            pltpu.async_copy(src_hbm.at[pl.ds(ebase + i * W, W)],
                             src_v[s], csem[s])
            pltpu.make_async_copy(dst_hbm.at[pl.ds(ebase, W)],
                                  dst_v[s], dsem[s]).wait()

            @plsc.parallel_loop(0, W, step=_LANES, unroll=8)
            def gbody(k):
                idx = dst_v[s][pl.ds(k, _LANES)]
                val_v[s][pl.ds(k, _LANES)] = plsc.load_gather(z_t, [idx])

            pltpu.make_async_copy(src_hbm.at[pl.ds(ebase, W)],
                                  src_v[s], csem[s]).wait()
            if not first:
                pltpu.make_async_copy(val_v[s], acc_sh.at[src_v[s]], ssem).wait()
            pltpu.async_copy(val_v[s], acc_sh.at[src_v[s]], ssem, add=True)
            @pl.when(i + 1 < NWIN)
            def _():
                fetch_dst(i + 1, 1 - s)

        fetch_dst(0, 0)
        window(0, 0, True)
        window(1, 1, False)

        def body(j, carry):
            window(2 * j, 0, False)
            window(2 * j + 1, 1, False)
            return carry

        lax.fori_loop(1, NWIN // 2, body, 0)

        pltpu.make_async_copy(val_v[1], acc_sh.at[src_v[1]], ssem).wait()

        plsc.subcore_barrier()

        @pl.when(sid == 0)
        def _writeout():
            pltpu.sync_copy(acc_sh.at[pl.ds(0, N)], out_hbm.at[cid])

    return seg(z, src, dst)


def _combine_body(s_ref, z_ref, d_ref, p_ref, o_ref):
    alpha = s_ref[0]
    beta = s_ref[1]
    gamma = s_ref[2]
    bias = s_ref[3]
    logd = jnp.log(d_ref[...])
    az = p_ref[0] + p_ref[1]
    o_ref[...] = (alpha * jnp.exp(gamma * logd) * z_ref[...]
                  + beta * jnp.exp((gamma - 1.0) * logd) * az + bias)


def kernel(z, edge_index, D, params):
    N = z.shape[0]
    E = edge_index.shape[1]
    src = edge_index[0]
    dst = edge_index[1]

    partial = _segment_partials(z, src, dst, N, E, 4000)

    alpha = jnp.exp(params[0])
    beta = -alpha * jnp.exp(params[1])
    gamma = jnp.exp(params[2])
    scal = jnp.stack([alpha, beta, gamma, params[3]])

    combine = pl.pallas_call(
        _combine_body,
        out_shape=jax.ShapeDtypeStruct((N,), jnp.float32),
        in_specs=[
            pl.BlockSpec(memory_space=pltpu.SMEM),
            pl.BlockSpec(memory_space=pltpu.VMEM),
            pl.BlockSpec(memory_space=pltpu.VMEM),
            pl.BlockSpec(memory_space=pltpu.VMEM),
        ],
        out_specs=pl.BlockSpec(memory_space=pltpu.VMEM),
    )
    return combine(scal, z, D, partial)


# speedup vs baseline: 1.7774x; 1.1141x over previous
import functools

import jax
import jax.numpy as jnp
from jax import lax
from jax.experimental import pallas as pl
from jax.experimental.pallas import tpu as pltpu
from jax.experimental.pallas import tpu_sc as plsc

_NC = 2
_NS = 16
_LANES = 16


@functools.partial(jax.jit, static_argnums=(2, 3, 4))
def _segment_partials(z, edge_index, N, E, W):
    NW = _NC * _NS
    EW = E // NW
    NWIN = EW // W
    assert EW * NW == E and NWIN * W == EW
    assert W % (2 * _LANES) == 0 and NWIN % 2 == 0 and NWIN >= 4
    SL = -(-N // (_NS * 8)) * 8
    NPAD = SL * _NS

    mesh = plsc.VectorSubcoreMesh(core_axis_name="c", subcore_axis_name="s")

    @functools.partial(
        pl.kernel,
        out_type=jax.ShapeDtypeStruct((_NC, N), jnp.float32),
        mesh=mesh,
        compiler_params=pltpu.CompilerParams(
            use_tc_tiling_on_sc=False, needs_layout_passes=False),
        scratch_types=[
            pltpu.VMEM_SHARED((NPAD,), jnp.float32),
            pltpu.VMEM((N,), jnp.float32),
            [pltpu.VMEM((W,), jnp.int32)] * 2,
            [pltpu.VMEM((W,), jnp.int32)] * 2,
            [pltpu.VMEM((W,), jnp.float32)] * 2,
            [pltpu.SemaphoreType.DMA] * 2,
            [pltpu.SemaphoreType.DMA] * 2,
            pltpu.SemaphoreType.DMA,
        ],
    )
    def seg(z_hbm, ei_hbm, out_hbm,
            acc_sh, z_t, dst_v, src_v, val_v, dsem, csem, ssem):
        cid = lax.axis_index("c")
        sid = lax.axis_index("s")
        ebase = (cid * _NS + sid) * EW

        def zbody(i, carry):
            val_v[0][pl.ds(i * _LANES, _LANES)] = jnp.zeros((_LANES,), jnp.float32)
            return carry

        lax.fori_loop(0, W // _LANES, zbody, 0)
        full, rem = divmod(SL, W)
        for q in range(full):
            pltpu.sync_copy(val_v[0], acc_sh.at[pl.ds(sid * SL + q * W, W)])
        if rem:
            pltpu.sync_copy(val_v[0].at[pl.ds(0, rem)],
                            acc_sh.at[pl.ds(sid * SL + full * W, rem)])

        pltpu.sync_copy(z_hbm, z_t)

        plsc.subcore_barrier()

        def fetch_dst(i, s):
            pltpu.async_copy(ei_hbm.at[1, pl.ds(ebase + i * W, W)],
                             dst_v[s], dsem[s])

        def window(i, s, first):
            pltpu.async_copy(ei_hbm.at[0, pl.ds(ebase + i * W, W)],
                             src_v[s], csem[s])
            pltpu.make_async_copy(ei_hbm.at[1, pl.ds(ebase, W)],
                                  dst_v[s], dsem[s]).wait()

            @plsc.parallel_loop(0, W, step=_LANES, unroll=8)
            def gbody(k):
                idx = dst_v[s][pl.ds(k, _LANES)]
                val_v[s][pl.ds(k, _LANES)] = plsc.load_gather(z_t, [idx])

            pltpu.make_async_copy(ei_hbm.at[0, pl.ds(ebase, W)],
                                  src_v[s], csem[s]).wait()
            if not first:
                pltpu.make_async_copy(val_v[s], acc_sh.at[src_v[s]], ssem).wait()
            pltpu.async_copy(val_v[s], acc_sh.at[src_v[s]], ssem, add=True)
            @pl.when(i + 1 < NWIN)
            def _():
                fetch_dst(i + 1, 1 - s)

        fetch_dst(0, 0)
        window(0, 0, True)
        window(1, 1, False)

        def body(j, carry):
            window(2 * j, 0, False)
            window(2 * j + 1, 1, False)
            return carry

        lax.fori_loop(1, NWIN // 2, body, 0)

        pltpu.make_async_copy(val_v[1], acc_sh.at[src_v[1]], ssem).wait()

        plsc.subcore_barrier()

        @pl.when(sid == 0)
        def _writeout():
            pltpu.sync_copy(acc_sh.at[pl.ds(0, N)], out_hbm.at[cid])

    return seg(z, edge_index)


def _combine_body(s_ref, z_ref, d_ref, p_ref, o_ref):
    alpha = s_ref[0]
    beta = s_ref[1]
    gamma = s_ref[2]
    bias = s_ref[3]
    logd = jnp.log(d_ref[...])
    az = p_ref[0] + p_ref[1]
    o_ref[...] = (alpha * jnp.exp(gamma * logd) * z_ref[...]
                  + beta * jnp.exp((gamma - 1.0) * logd) * az + bias)


def kernel(z, edge_index, D, params):
    N = z.shape[0]
    E = edge_index.shape[1]

    partial = _segment_partials(z, edge_index, N, E, 4000)

    alpha = jnp.exp(params[0])
    beta = -alpha * jnp.exp(params[1])
    gamma = jnp.exp(params[2])
    scal = jnp.stack([alpha, beta, gamma, params[3]])

    combine = pl.pallas_call(
        _combine_body,
        out_shape=jax.ShapeDtypeStruct((N,), jnp.float32),
        in_specs=[
            pl.BlockSpec(memory_space=pltpu.SMEM),
            pl.BlockSpec(memory_space=pltpu.VMEM),
            pl.BlockSpec(memory_space=pltpu.VMEM),
            pl.BlockSpec(memory_space=pltpu.VMEM),
        ],
        out_specs=pl.BlockSpec(memory_space=pltpu.VMEM),
    )
    return combine(scal, z, D, partial)


# speedup vs baseline: 2.2056x; 1.2409x over previous
import functools

import jax
import jax.numpy as jnp
from jax import lax
from jax.experimental import pallas as pl
from jax.experimental.pallas import tpu as pltpu
from jax.experimental.pallas import tpu_sc as plsc

_NC = 2
_NS = 16
_LANES = 16
_W = 2048


def _npad(N):
    return (-(-N // (_NS * 128)) * 128) * _NS


@functools.partial(jax.jit, static_argnums=(2, 3))
def _segment_partials(z, edge_index, N, E):
    NW = _NC * _NS
    G = E // _W
    assert G * _W == E
    NFULL, NEXTRA = divmod(G, NW)
    NPAD = _npad(N)
    SL = NPAD // _NS

    mesh = plsc.VectorSubcoreMesh(core_axis_name="c", subcore_axis_name="s")

    @functools.partial(
        pl.kernel,
        out_type=jax.ShapeDtypeStruct((_NC * NPAD,), jnp.float32),
        mesh=mesh,
        compiler_params=pltpu.CompilerParams(
            use_tc_tiling_on_sc=True, needs_layout_passes=False),
        scratch_types=[
            pltpu.VMEM_SHARED((NPAD,), jnp.float32),
            pltpu.VMEM((N,), jnp.float32),
            [pltpu.VMEM((2, _W), jnp.int32)] * 2,
            [pltpu.VMEM((_W,), jnp.int32)] * 2,
            [pltpu.VMEM((_W,), jnp.float32)] * 2,
            [pltpu.SemaphoreType.DMA] * 2,
            pltpu.SemaphoreType.DMA,
        ],
    )
    def seg(z_hbm, ei_hbm, out_hbm,
            acc_sh, z_t, ei_v, src_v, val_v, esem, ssem):
        cid = lax.axis_index("c")
        sid = lax.axis_index("s")
        wid = cid * _NS + sid
        nwin = jnp.where(wid < NEXTRA, NFULL + 1, NFULL)

        def zbody(i, carry):
            val_v[0][pl.ds(i * _LANES, _LANES)] = jnp.zeros((_LANES,), jnp.float32)
            return carry

        lax.fori_loop(0, _W // _LANES, zbody, 0)
        full, rem = divmod(SL, _W)
        for q in range(full):
            pltpu.sync_copy(val_v[0], acc_sh.at[pl.ds(sid * SL + q * _W, _W)])
        if rem:
            pltpu.sync_copy(val_v[0].at[pl.ds(0, rem)],
                            acc_sh.at[pl.ds(sid * SL + full * _W, rem)])

        pltpu.sync_copy(z_hbm, z_t)

        plsc.subcore_barrier()

        def fetch(i, s):
            off = (wid + i * NW) * _W
            pltpu.async_copy(ei_hbm.at[:, pl.ds(off, _W)], ei_v[s], esem[s])

        def window(i, s, first):
            pltpu.make_async_copy(ei_hbm.at[:, pl.ds(0, _W)],
                                  ei_v[s], esem[s]).wait()
            @pl.when(i + 1 < nwin)
            def _():
                fetch(i + 1, 1 - s)

            @plsc.parallel_loop(0, _W, step=_LANES, unroll=8)
            def gbody(k):
                src_v[s][pl.ds(k, _LANES)] = ei_v[s][0, pl.ds(k, _LANES)]
                idx = ei_v[s][1, pl.ds(k, _LANES)]
                val_v[s][pl.ds(k, _LANES)] = plsc.load_gather(z_t, [idx])

            if not first:
                pltpu.make_async_copy(val_v[s], acc_sh.at[src_v[s]], ssem).wait()
            pltpu.async_copy(val_v[s], acc_sh.at[src_v[s]], ssem, add=True)

        fetch(0, 0)
        window(0, 0, True)
        window(1, 1, False)

        def body(j, carry):
            window(2 * j, 0, False)
            window(2 * j + 1, 1, False)
            return carry

        lax.fori_loop(1, nwin // 2, body, 0)

        @pl.when(nwin % 2 == 1)
        def _odd_tail():
            pltpu.make_async_copy(ei_hbm.at[:, pl.ds(0, _W)],
                                  ei_v[0], esem[0]).wait()

            @plsc.parallel_loop(0, _W, step=_LANES, unroll=8)
            def gbody(k):
                src_v[0][pl.ds(k, _LANES)] = ei_v[0][0, pl.ds(k, _LANES)]
                idx = ei_v[0][1, pl.ds(k, _LANES)]
                val_v[0][pl.ds(k, _LANES)] = plsc.load_gather(z_t, [idx])

            pltpu.make_async_copy(val_v[0], acc_sh.at[src_v[0]], ssem).wait()
            pltpu.async_copy(val_v[0], acc_sh.at[src_v[0]], ssem, add=True)

        pltpu.make_async_copy(val_v[0], acc_sh.at[src_v[0]], ssem).wait()

        plsc.subcore_barrier()

        @pl.when(sid == 0)
        def _writeout():
            pltpu.sync_copy(acc_sh, out_hbm.at[pl.ds(cid * NPAD, NPAD)])

    return seg(z, edge_index)


def _make_combine(N, NPAD):
    def body(s_ref, z_ref, d_ref, p_ref, o_ref):
        alpha = s_ref[0]
        beta = s_ref[1]
        gamma = s_ref[2]
        bias = s_ref[3]
        logd = jnp.log(d_ref[...])
        az = p_ref[pl.ds(0, N)] + p_ref[pl.ds(NPAD, N)]
        o_ref[...] = (alpha * jnp.exp(gamma * logd) * z_ref[...]
                      + beta * jnp.exp((gamma - 1.0) * logd) * az + bias)

    return pl.pallas_call(
        body,
        out_shape=jax.ShapeDtypeStruct((N,), jnp.float32),
        in_specs=[
            pl.BlockSpec(memory_space=pltpu.SMEM),
            pl.BlockSpec(memory_space=pltpu.VMEM),
            pl.BlockSpec(memory_space=pltpu.VMEM),
            pl.BlockSpec(memory_space=pltpu.VMEM),
        ],
        out_specs=pl.BlockSpec(memory_space=pltpu.VMEM),
    )


def kernel(z, edge_index, D, params):
    N = z.shape[0]
    E = edge_index.shape[1]

    partial = _segment_partials(z, edge_index, N, E)
    NPAD = _npad(N)

    alpha = jnp.exp(params[0])
    beta = -alpha * jnp.exp(params[1])
    gamma = jnp.exp(params[2])
    scal = jnp.stack([alpha, beta, gamma, params[3]])

    return _make_combine(N, NPAD)(scal, z, D, partial)


# speedup vs baseline: 2.4402x; 1.1063x over previous
import functools

import jax
import jax.numpy as jnp
from jax import lax
from jax.experimental import pallas as pl
from jax.experimental.pallas import tpu as pltpu
from jax.experimental.pallas import tpu_sc as plsc

_NC = 2
_NS = 16
_LANES = 16
_W = 2560


def _npad(N):
    return (-(-N // (_NS * 128)) * 128) * _NS


@functools.partial(jax.jit, static_argnums=(2, 3))
def _segment_partials(z, edge_index, N, E):
    NW = _NC * _NS
    G = E // _W
    assert G * _W == E
    NFULL, NEXTRA = divmod(G, NW)
    NPAD = _npad(N)
    SL = NPAD // _NS

    mesh = plsc.VectorSubcoreMesh(core_axis_name="c", subcore_axis_name="s")

    @functools.partial(
        pl.kernel,
        out_type=jax.ShapeDtypeStruct((_NC * NPAD,), jnp.float32),
        mesh=mesh,
        compiler_params=pltpu.CompilerParams(
            use_tc_tiling_on_sc=True, needs_layout_passes=False),
        scratch_types=[
            pltpu.VMEM_SHARED((NPAD,), jnp.float32),
            pltpu.VMEM((N,), jnp.float32),
            [pltpu.VMEM((2, _W), jnp.int32)] * 2,
            [pltpu.VMEM((_W,), jnp.int32)] * 2,
            [pltpu.VMEM((_W,), jnp.float32)] * 2,
            [pltpu.SemaphoreType.DMA] * 2,
            pltpu.SemaphoreType.DMA,
        ],
    )
    def seg(z_hbm, ei_hbm, out_hbm,
            acc_sh, z_t, ei_v, src_v, val_v, esem, ssem):
        cid = lax.axis_index("c")
        sid = lax.axis_index("s")
        wid = cid * _NS + sid
        nwin = jnp.where(wid < NEXTRA, NFULL + 1, NFULL)

        def zbody(i, carry):
            val_v[0][pl.ds(i * _LANES, _LANES)] = jnp.zeros((_LANES,), jnp.float32)
            return carry

        lax.fori_loop(0, _W // _LANES, zbody, 0)
        full, rem = divmod(SL, _W)
        for q in range(full):
            pltpu.sync_copy(val_v[0], acc_sh.at[pl.ds(sid * SL + q * _W, _W)])
        if rem:
            pltpu.sync_copy(val_v[0].at[pl.ds(0, rem)],
                            acc_sh.at[pl.ds(sid * SL + full * _W, rem)])

        pltpu.sync_copy(z_hbm, z_t)

        plsc.subcore_barrier()

        def fetch(i, s):
            off = (wid + i * NW) * _W
            pltpu.async_copy(ei_hbm.at[:, pl.ds(off, _W)], ei_v[s], esem[s])

        def window(i, s, first):
            pltpu.make_async_copy(ei_hbm.at[:, pl.ds(0, _W)],
                                  ei_v[s], esem[s]).wait()
            @pl.when(i + 1 < nwin)
            def _():
                fetch(i + 1, 1 - s)

            @plsc.parallel_loop(0, _W, step=_LANES, unroll=8)
            def gbody(k):
                src_v[s][pl.ds(k, _LANES)] = ei_v[s][0, pl.ds(k, _LANES)]
                idx = ei_v[s][1, pl.ds(k, _LANES)]
                val_v[s][pl.ds(k, _LANES)] = plsc.load_gather(z_t, [idx])

            if not first:
                pltpu.make_async_copy(val_v[s], acc_sh.at[src_v[s]], ssem).wait()
            pltpu.async_copy(val_v[s], acc_sh.at[src_v[s]], ssem, add=True)

        fetch(0, 0)
        window(0, 0, True)
        window(1, 1, False)

        def body(j, carry):
            window(2 * j, 0, False)
            window(2 * j + 1, 1, False)
            return carry

        lax.fori_loop(1, nwin // 2, body, 0)

        @pl.when(nwin % 2 == 1)
        def _odd_tail():
            pltpu.make_async_copy(ei_hbm.at[:, pl.ds(0, _W)],
                                  ei_v[0], esem[0]).wait()

            @plsc.parallel_loop(0, _W, step=_LANES, unroll=8)
            def gbody(k):
                src_v[0][pl.ds(k, _LANES)] = ei_v[0][0, pl.ds(k, _LANES)]
                idx = ei_v[0][1, pl.ds(k, _LANES)]
                val_v[0][pl.ds(k, _LANES)] = plsc.load_gather(z_t, [idx])

            pltpu.make_async_copy(val_v[0], acc_sh.at[src_v[0]], ssem).wait()
            pltpu.async_copy(val_v[0], acc_sh.at[src_v[0]], ssem, add=True)

        pltpu.make_async_copy(val_v[0], acc_sh.at[src_v[0]], ssem).wait()

        plsc.subcore_barrier()

        @pl.when(sid == 0)
        def _writeout():
            pltpu.sync_copy(acc_sh, out_hbm.at[pl.ds(cid * NPAD, NPAD)])

    return seg(z, edge_index)


def _make_combine(N, NPAD):
    def body(s_ref, z_ref, d_ref, p_ref, o_ref):
        alpha = s_ref[0]
        beta = s_ref[1]
        gamma = s_ref[2]
        bias = s_ref[3]
        logd = jnp.log(d_ref[...])
        az = p_ref[pl.ds(0, N)] + p_ref[pl.ds(NPAD, N)]
        o_ref[...] = (alpha * jnp.exp(gamma * logd) * z_ref[...]
                      + beta * jnp.exp((gamma - 1.0) * logd) * az + bias)

    return pl.pallas_call(
        body,
        out_shape=jax.ShapeDtypeStruct((N,), jnp.float32),
        in_specs=[
            pl.BlockSpec(memory_space=pltpu.SMEM),
            pl.BlockSpec(memory_space=pltpu.VMEM),
            pl.BlockSpec(memory_space=pltpu.VMEM),
            pl.BlockSpec(memory_space=pltpu.VMEM),
        ],
        out_specs=pl.BlockSpec(memory_space=pltpu.VMEM),
    )


def kernel(z, edge_index, D, params):
    N = z.shape[0]
    E = edge_index.shape[1]

    partial = _segment_partials(z, edge_index, N, E)
    NPAD = _npad(N)

    alpha = jnp.exp(params[0])
    beta = -alpha * jnp.exp(params[1])
    gamma = jnp.exp(params[2])
    scal = jnp.stack([alpha, beta, gamma, params[3]])

    return _make_combine(N, NPAD)(scal, z, D, partial)


# speedup vs baseline: 2.5226x; 1.0338x over previous
import functools

import jax
import jax.numpy as jnp
from jax import lax
from jax.experimental import pallas as pl
from jax.experimental.pallas import tpu as pltpu
from jax.experimental.pallas import tpu_sc as plsc

_NC = 2
_NS = 16
_LANES = 16
_W = 2560


def _npad(N):
    return (-(-N // (_NS * 128)) * 128) * _NS


@functools.partial(jax.jit, static_argnums=(2, 3))
def _segment_partials(z, edge_index, N, E):
    NW = _NC * _NS
    G = E // _W
    assert G * _W == E
    NFULL, NEXTRA = divmod(G, NW)
    NPAD = _npad(N)
    SL = NPAD // _NS

    mesh = plsc.VectorSubcoreMesh(core_axis_name="c", subcore_axis_name="s")

    @functools.partial(
        pl.kernel,
        out_type=jax.ShapeDtypeStruct((_NC * NPAD,), jnp.float32),
        mesh=mesh,
        compiler_params=pltpu.CompilerParams(
            use_tc_tiling_on_sc=True, needs_layout_passes=False),
        scratch_types=[
            pltpu.VMEM_SHARED((NPAD,), jnp.float32),
            pltpu.VMEM((N,), jnp.float32),
            [pltpu.VMEM((2, _W), jnp.int32)] * 2,
            [pltpu.VMEM((_W,), jnp.int32)] * 2,
            [pltpu.VMEM((_W,), jnp.float32)] * 2,
            [pltpu.SemaphoreType.DMA] * 2,
            pltpu.SemaphoreType.DMA,
        ],
    )
    def seg(z_hbm, ei_hbm, out_hbm,
            acc_sh, z_t, ei_v, src_v, val_v, esem, ssem):
        cid = lax.axis_index("c")
        sid = lax.axis_index("s")
        wid = cid * _NS + sid
        nwin = jnp.where(wid < NEXTRA, NFULL + 1, NFULL)

        def fetch(i, s):
            off = (wid + i * NW) * _W
            pltpu.async_copy(ei_hbm.at[:, pl.ds(off, _W)], ei_v[s], esem[s])

        fetch(0, 0)
        fetch(1, 1)

        def zbody(i, carry):
            val_v[0][pl.ds(i * _LANES, _LANES)] = jnp.zeros((_LANES,), jnp.float32)
            return carry

        lax.fori_loop(0, _W // _LANES, zbody, 0)
        full, rem = divmod(SL, _W)
        for q in range(full):
            pltpu.sync_copy(val_v[0], acc_sh.at[pl.ds(sid * SL + q * _W, _W)])
        if rem:
            pltpu.sync_copy(val_v[0].at[pl.ds(0, rem)],
                            acc_sh.at[pl.ds(sid * SL + full * _W, rem)])

        pltpu.sync_copy(z_hbm, z_t)

        plsc.subcore_barrier()

        def window(i, s, first):
            pltpu.make_async_copy(ei_hbm.at[:, pl.ds(0, _W)],
                                  ei_v[s], esem[s]).wait()

            @plsc.parallel_loop(0, _W, step=_LANES, unroll=8)
            def gbody(k):
                src_v[s][pl.ds(k, _LANES)] = ei_v[s][0, pl.ds(k, _LANES)]
                idx = ei_v[s][1, pl.ds(k, _LANES)]
                val_v[s][pl.ds(k, _LANES)] = plsc.load_gather(z_t, [idx])

            @pl.when(i + 2 < nwin)
            def _():
                fetch(i + 2, s)

            if not first:
                pltpu.make_async_copy(val_v[s], acc_sh.at[src_v[s]], ssem).wait()
            pltpu.async_copy(val_v[s], acc_sh.at[src_v[s]], ssem, add=True)

        window(0, 0, True)
        window(1, 1, False)

        def body(j, carry):
            window(2 * j, 0, False)
            window(2 * j + 1, 1, False)
            return carry

        lax.fori_loop(1, nwin // 2, body, 0)

        @pl.when(nwin % 2 == 1)
        def _odd_tail():
            pltpu.make_async_copy(ei_hbm.at[:, pl.ds(0, _W)],
                                  ei_v[0], esem[0]).wait()

            @plsc.parallel_loop(0, _W, step=_LANES, unroll=8)
            def gbody(k):
                src_v[0][pl.ds(k, _LANES)] = ei_v[0][0, pl.ds(k, _LANES)]
                idx = ei_v[0][1, pl.ds(k, _LANES)]
                val_v[0][pl.ds(k, _LANES)] = plsc.load_gather(z_t, [idx])

            pltpu.make_async_copy(val_v[0], acc_sh.at[src_v[0]], ssem).wait()
            pltpu.async_copy(val_v[0], acc_sh.at[src_v[0]], ssem, add=True)

        pltpu.make_async_copy(val_v[0], acc_sh.at[src_v[0]], ssem).wait()

        plsc.subcore_barrier()

        @pl.when(sid == 0)
        def _writeout():
            pltpu.sync_copy(acc_sh, out_hbm.at[pl.ds(cid * NPAD, NPAD)])

    return seg(z, edge_index)


def _make_combine(N, NPAD):
    def body(s_ref, z_ref, d_ref, p_ref, o_ref):
        alpha = s_ref[0]
        beta = s_ref[1]
        gamma = s_ref[2]
        bias = s_ref[3]
        logd = jnp.log(d_ref[...])
        az = p_ref[pl.ds(0, N)] + p_ref[pl.ds(NPAD, N)]
        o_ref[...] = (alpha * jnp.exp(gamma * logd) * z_ref[...]
                      + beta * jnp.exp((gamma - 1.0) * logd) * az + bias)

    return pl.pallas_call(
        body,
        out_shape=jax.ShapeDtypeStruct((N,), jnp.float32),
        in_specs=[
            pl.BlockSpec(memory_space=pltpu.SMEM),
            pl.BlockSpec(memory_space=pltpu.VMEM),
            pl.BlockSpec(memory_space=pltpu.VMEM),
            pl.BlockSpec(memory_space=pltpu.VMEM),
        ],
        out_specs=pl.BlockSpec(memory_space=pltpu.VMEM),
    )


def kernel(z, edge_index, D, params):
    N = z.shape[0]
    E = edge_index.shape[1]

    partial = _segment_partials(z, edge_index, N, E)
    NPAD = _npad(N)

    alpha = jnp.exp(params[0])
    beta = -alpha * jnp.exp(params[1])
    gamma = jnp.exp(params[2])
    scal = jnp.stack([alpha, beta, gamma, params[3]])

    return _make_combine(N, NPAD)(scal, z, D, partial)


# speedup vs baseline: 2.6195x; 1.0384x over previous
import functools

import jax
import jax.numpy as jnp
from jax import lax
from jax.experimental import pallas as pl
from jax.experimental.pallas import tpu as pltpu
from jax.experimental.pallas import tpu_sc as plsc

_NC = 2
_NS = 16
_LANES = 16
_W = 2560


def _npad(N):
    return (-(-N // (_NS * 128)) * 128) * _NS


@functools.partial(jax.jit, static_argnums=(2, 3))
def _segment_partials(z, edge_index, N, E):
    NW = _NC * _NS
    G = E // _W
    assert G * _W == E
    NFULL, NEXTRA = divmod(G, NW)
    NPAD = _npad(N)
    SL = NPAD // _NS

    mesh = plsc.VectorSubcoreMesh(core_axis_name="c", subcore_axis_name="s")

    @functools.partial(
        pl.kernel,
        out_type=jax.ShapeDtypeStruct((_NC * NPAD,), jnp.float32),
        mesh=mesh,
        compiler_params=pltpu.CompilerParams(
            use_tc_tiling_on_sc=True, needs_layout_passes=False),
        scratch_types=[
            pltpu.VMEM_SHARED((NPAD,), jnp.float32),
            pltpu.VMEM((N,), jnp.float32),
            [pltpu.VMEM((2, _W), jnp.int32)] * 2,
            [pltpu.VMEM((_W,), jnp.int32)] * 2,
            [pltpu.VMEM((_W,), jnp.float32)] * 2,
            [pltpu.SemaphoreType.DMA] * 2,
            [pltpu.SemaphoreType.DMA] * 2,
        ],
    )
    def seg(z_hbm, ei_hbm, out_hbm,
            acc_sh, z_t, ei_v, src_v, val_v, esem, ssem):
        cid = lax.axis_index("c")
        sid = lax.axis_index("s")
        wid = cid * _NS + sid
        nwin = jnp.where(wid < NEXTRA, NFULL + 1, NFULL)

        def fetch(i, s):
            off = (wid + i * NW) * _W
            pltpu.async_copy(ei_hbm.at[:, pl.ds(off, _W)], ei_v[s], esem[s])

        fetch(0, 0)
        fetch(1, 1)

        def zbody(i, carry):
            val_v[0][pl.ds(i * _LANES, _LANES)] = jnp.zeros((_LANES,), jnp.float32)
            return carry

        lax.fori_loop(0, _W // _LANES, zbody, 0)
        full, rem = divmod(SL, _W)
        for q in range(full):
            pltpu.sync_copy(val_v[0], acc_sh.at[pl.ds(sid * SL + q * _W, _W)])
        if rem:
            pltpu.sync_copy(val_v[0].at[pl.ds(0, rem)],
                            acc_sh.at[pl.ds(sid * SL + full * _W, rem)])

        pltpu.sync_copy(z_hbm, z_t)

        plsc.subcore_barrier()

        def window(i, s, first):
            pltpu.make_async_copy(ei_hbm.at[:, pl.ds(0, _W)],
                                  ei_v[s], esem[s]).wait()
            if not first:
                pltpu.make_async_copy(val_v[s], acc_sh.at[src_v[s]],
                                      ssem[s]).wait()

            @plsc.parallel_loop(0, _W, step=_LANES, unroll=16)
            def gbody(k):
                src_v[s][pl.ds(k, _LANES)] = ei_v[s][0, pl.ds(k, _LANES)]
                idx = ei_v[s][1, pl.ds(k, _LANES)]
                val_v[s][pl.ds(k, _LANES)] = plsc.load_gather(z_t, [idx])

            @pl.when(i + 2 < nwin)
            def _():
                fetch(i + 2, s)

            pltpu.async_copy(val_v[s], acc_sh.at[src_v[s]], ssem[s], add=True)

        window(0, 0, True)
        window(1, 1, True)

        def body(j, carry):
            window(2 * j, 0, False)
            window(2 * j + 1, 1, False)
            return carry

        lax.fori_loop(1, nwin // 2, body, 0)

        @pl.when(nwin % 2 == 1)
        def _odd_tail():
            pltpu.make_async_copy(ei_hbm.at[:, pl.ds(0, _W)],
                                  ei_v[0], esem[0]).wait()
            pltpu.make_async_copy(val_v[0], acc_sh.at[src_v[0]], ssem[0]).wait()

            @plsc.parallel_loop(0, _W, step=_LANES, unroll=16)
            def gbody(k):
                src_v[0][pl.ds(k, _LANES)] = ei_v[0][0, pl.ds(k, _LANES)]
                idx = ei_v[0][1, pl.ds(k, _LANES)]
                val_v[0][pl.ds(k, _LANES)] = plsc.load_gather(z_t, [idx])

            pltpu.async_copy(val_v[0], acc_sh.at[src_v[0]], ssem[0], add=True)

        for h in range(2):
            pltpu.make_async_copy(val_v[h], acc_sh.at[src_v[h]], ssem[h]).wait()

        plsc.subcore_barrier()

        @pl.when(sid == 0)
        def _writeout():
            pltpu.sync_copy(acc_sh, out_hbm.at[pl.ds(cid * NPAD, NPAD)])

    return seg(z, edge_index)


def _make_combine(N, NPAD):
    def body(s_ref, z_ref, d_ref, p_ref, o_ref):
        alpha = s_ref[0]
        beta = s_ref[1]
        gamma = s_ref[2]
        bias = s_ref[3]
        logd = jnp.log(d_ref[...])
        az = p_ref[pl.ds(0, N)] + p_ref[pl.ds(NPAD, N)]
        o_ref[...] = (alpha * jnp.exp(gamma * logd) * z_ref[...]
                      + beta * jnp.exp((gamma - 1.0) * logd) * az + bias)

    return pl.pallas_call(
        body,
        out_shape=jax.ShapeDtypeStruct((N,), jnp.float32),
        in_specs=[
            pl.BlockSpec(memory_space=pltpu.SMEM),
            pl.BlockSpec(memory_space=pltpu.VMEM),
            pl.BlockSpec(memory_space=pltpu.VMEM),
            pl.BlockSpec(memory_space=pltpu.VMEM),
        ],
        out_specs=pl.BlockSpec(memory_space=pltpu.VMEM),
    )


def kernel(z, edge_index, D, params):
    N = z.shape[0]
    E = edge_index.shape[1]

    partial = _segment_partials(z, edge_index, N, E)
    NPAD = _npad(N)

    alpha = jnp.exp(params[0])
    beta = -alpha * jnp.exp(params[1])
    gamma = jnp.exp(params[2])
    scal = jnp.stack([alpha, beta, gamma, params[3]])

    return _make_combine(N, NPAD)(scal, z, D, partial)
